# baseline (speedup 1.0000x reference)
import functools

import jax
import jax.numpy as jnp
from jax.experimental import pallas as pl
from jax.experimental.pallas import tpu as pltpu

_MC = 1024
_MXU_FRAC = 0.75


def _bf16_round(v):
    return v.astype(jnp.bfloat16).astype(jnp.float32)


def _cd_kernel(xr, x0r, x1r, x2r, y0r, y1r, y2r, d1r, d2r, sr, *, m, mc,
               n_mxu_chunks, inv1, inv2):
    x = xr[0]
    x0 = x0r[0]
    x1 = x1r[0]
    x2 = x2r[0]
    y0 = y0r[0]
    y1 = y1r[0]
    y2 = y2r[0]

    sq1 = x0 * x0 + x1 * x1 + x2 * x2
    sq2 = y0 * y0 + y1 * y1 + y2 * y2

    s1h = _bf16_round(sq1)
    s1m = _bf16_round(sq1 - s1h)
    s1l = sq1 - s1h - s1m
    s2h = _bf16_round(sq2)
    s2m = _bf16_round(sq2 - s2h)
    s2l = sq2 - s2h - s2m
    onex = jnp.ones_like(x)
    x9 = jnp.concatenate([x * -2.0, onex, s1h, s1m, s1l],
                         axis=1).astype(jnp.bfloat16)
    y9 = jnp.concatenate(
        [y0, y1, y2, s2h, s2m, s2l,
         jnp.ones((3,) + y0.shape[1:], jnp.float32)],
        axis=0).astype(jnp.bfloat16)
    u0 = _bf16_round(x0) * -2.0
    u1 = _bf16_round(x1) * -2.0
    u2 = _bf16_round(x2) * -2.0
    v0 = _bf16_round(y0)
    v1 = _bf16_round(y1)
    v2 = _bf16_round(y2)

    d1run = None
    s2 = jnp.zeros((1, 1), jnp.float32)
    for c in range(m // mc):
        lo, hi = c * mc, (c + 1) * mc
        if c < n_mxu_chunks:
            d = jax.lax.dot_general(
                x9, y9[:, lo:hi], (((1,), (0,)), ((), ())),
                preferred_element_type=jnp.float32)
        else:
            d = (sq1 + sq2[:, lo:hi]) + u0 * v0[:, lo:hi]
            d = d + u1 * v1[:, lo:hi]
            d = d + u2 * v2[:, lo:hi]
        cmin = jnp.maximum(jnp.min(d, axis=0, keepdims=True), 0.0)
        d2r[0, :, lo:hi] = cmin
        s2 = s2 + jnp.sum(cmin, axis=1, keepdims=True)
        d1run = d if d1run is None else jnp.minimum(d1run, d)

    d1 = jnp.maximum(jnp.min(d1run, axis=1, keepdims=True), 0.0)
    d1r[0] = d1
    sr[0] = jnp.sum(d1, axis=0, keepdims=True) * inv1 + s2 * inv2


def kernel(pcs1, pcs2):
    B, N, D = pcs1.shape
    M = pcs2.shape[1]
    assert D == 3
    f32 = jnp.float32

    x0 = pcs1[:, :, 0:1]
    x1 = pcs1[:, :, 1:2]
    x2 = pcs1[:, :, 2:3]
    y0 = pcs2[:, :, 0][:, None, :]
    y1 = pcs2[:, :, 1][:, None, :]
    y2 = pcs2[:, :, 2][:, None, :]

    n_chunks = M // _MC
    n_mxu = int(round(_MXU_FRAC * n_chunks))
    body = functools.partial(_cd_kernel, m=M, mc=_MC, n_mxu_chunks=n_mxu,
                             inv1=1.0 / (B * N), inv2=1.0 / (B * M))

    d1, d2, psum = pl.pallas_call(
        body,
        grid=(B,),
        in_specs=[
            pl.BlockSpec((1, N, 3), lambda i: (i, 0, 0)),
            pl.BlockSpec((1, N, 1), lambda i: (i, 0, 0)),
            pl.BlockSpec((1, N, 1), lambda i: (i, 0, 0)),
            pl.BlockSpec((1, N, 1), lambda i: (i, 0, 0)),
            pl.BlockSpec((1, 1, M), lambda i: (i, 0, 0)),
            pl.BlockSpec((1, 1, M), lambda i: (i, 0, 0)),
            pl.BlockSpec((1, 1, M), lambda i: (i, 0, 0)),
        ],
        out_specs=[
            pl.BlockSpec((1, N, 1), lambda i: (i, 0, 0)),
            pl.BlockSpec((1, 1, M), lambda i: (i, 0, 0)),
            pl.BlockSpec((1, 1, 1), lambda i: (i, 0, 0)),
        ],
        out_shape=[
            jax.ShapeDtypeStruct((B, N, 1), f32),
            jax.ShapeDtypeStruct((B, 1, M), f32),
            jax.ShapeDtypeStruct((B, 1, 1), f32),
        ],
        compiler_params=pltpu.CompilerParams(
            dimension_semantics=("parallel",)),
    )(pcs1, x0, x1, x2, y0, y1, y2)

    mean = jnp.sum(psum)
    return mean, d1[:, :, 0], d2[:, 0, :]


# speedup vs baseline: 1.0998x; 1.0998x over previous
import functools

import jax
import jax.numpy as jnp
from jax.experimental import pallas as pl
from jax.experimental.pallas import tpu as pltpu

_MC = 1024
_MXU_FRAC = 1.0


def _bf16_round(v):
    return v.astype(jnp.bfloat16).astype(jnp.float32)


def _cd_kernel(xr, x0r, x1r, x2r, y0r, y1r, y2r, d1r, d2r, sr, *, m, mc,
               n_mxu_chunks, inv1, inv2):
    x = xr[0]
    x0 = x0r[0]
    x1 = x1r[0]
    x2 = x2r[0]
    y0 = y0r[0]
    y1 = y1r[0]
    y2 = y2r[0]

    sq1 = x0 * x0 + x1 * x1 + x2 * x2
    sq2 = y0 * y0 + y1 * y1 + y2 * y2

    s1h = _bf16_round(sq1)
    s1m = _bf16_round(sq1 - s1h)
    s1l = sq1 - s1h - s1m
    s2h = _bf16_round(sq2)
    s2m = _bf16_round(sq2 - s2h)
    s2l = sq2 - s2h - s2m
    onex = jnp.ones_like(x)
    x9 = jnp.concatenate([x * -2.0, onex, s1h, s1m, s1l],
                         axis=1).astype(jnp.bfloat16)
    y9 = jnp.concatenate(
        [y0, y1, y2, s2h, s2m, s2l,
         jnp.ones((3,) + y0.shape[1:], jnp.float32)],
        axis=0).astype(jnp.bfloat16)
    u0 = _bf16_round(x0) * -2.0
    u1 = _bf16_round(x1) * -2.0
    u2 = _bf16_round(x2) * -2.0
    v0 = _bf16_round(y0)
    v1 = _bf16_round(y1)
    v2 = _bf16_round(y2)

    d1col = None
    s2 = jnp.zeros((1, 1), jnp.float32)
    for c in range(m // mc):
        lo, hi = c * mc, (c + 1) * mc
        if c < n_mxu_chunks:
            d = jax.lax.dot_general(
                x9, y9[:, lo:hi], (((1,), (0,)), ((), ())),
                preferred_element_type=jnp.float32)
        else:
            d = (sq1 + sq2[:, lo:hi]) + u0 * v0[:, lo:hi]
            d = d + u1 * v1[:, lo:hi]
            d = d + u2 * v2[:, lo:hi]
        cmin = jnp.maximum(jnp.min(d, axis=0, keepdims=True), 0.0)
        d2r[0, :, lo:hi] = cmin
        s2 = s2 + jnp.sum(cmin, axis=1, keepdims=True)
        rmin = jnp.min(d, axis=1, keepdims=True)
        d1col = rmin if d1col is None else jnp.minimum(d1col, rmin)

    d1 = jnp.maximum(d1col, 0.0)
    d1r[0] = d1
    sr[0] = jnp.sum(d1, axis=0, keepdims=True) * inv1 + s2 * inv2


def kernel(pcs1, pcs2):
    B, N, D = pcs1.shape
    M = pcs2.shape[1]
    assert D == 3
    f32 = jnp.float32

    x0 = pcs1[:, :, 0:1]
    x1 = pcs1[:, :, 1:2]
    x2 = pcs1[:, :, 2:3]
    y0 = pcs2[:, :, 0][:, None, :]
    y1 = pcs2[:, :, 1][:, None, :]
    y2 = pcs2[:, :, 2][:, None, :]

    n_chunks = M // _MC
    n_mxu = int(round(_MXU_FRAC * n_chunks))
    body = functools.partial(_cd_kernel, m=M, mc=_MC, n_mxu_chunks=n_mxu,
                             inv1=1.0 / (B * N), inv2=1.0 / (B * M))

    d1, d2, psum = pl.pallas_call(
        body,
        grid=(B,),
        in_specs=[
            pl.BlockSpec((1, N, 3), lambda i: (i, 0, 0)),
            pl.BlockSpec((1, N, 1), lambda i: (i, 0, 0)),
            pl.BlockSpec((1, N, 1), lambda i: (i, 0, 0)),
            pl.BlockSpec((1, N, 1), lambda i: (i, 0, 0)),
            pl.BlockSpec((1, 1, M), lambda i: (i, 0, 0)),
            pl.BlockSpec((1, 1, M), lambda i: (i, 0, 0)),
            pl.BlockSpec((1, 1, M), lambda i: (i, 0, 0)),
        ],
        out_specs=[
            pl.BlockSpec((1, N, 1), lambda i: (i, 0, 0)),
            pl.BlockSpec((1, 1, M), lambda i: (i, 0, 0)),
            pl.BlockSpec((1, 1, 1), lambda i: (i, 0, 0)),
        ],
        out_shape=[
            jax.ShapeDtypeStruct((B, N, 1), f32),
            jax.ShapeDtypeStruct((B, 1, M), f32),
            jax.ShapeDtypeStruct((B, 1, 1), f32),
        ],
        compiler_params=pltpu.CompilerParams(
            dimension_semantics=("parallel",)),
    )(pcs1, x0, x1, x2, y0, y1, y2)

    mean = jnp.sum(psum)
    return mean, d1[:, :, 0], d2[:, 0, :]


# speedup vs baseline: 1.0998x; 1.0000x over previous
import functools

import jax
import jax.numpy as jnp
from jax.experimental import pallas as pl
from jax.experimental.pallas import tpu as pltpu

_MC = 2048
_MXU_FRAC = 1.0


def _bf16_round(v):
    return v.astype(jnp.bfloat16).astype(jnp.float32)


def _cd_kernel(xr, x0r, x1r, x2r, y0r, y1r, y2r, d1r, d2r, sr, *, m, mc,
               n_mxu_chunks, inv1, inv2):
    x = xr[0]
    x0 = x0r[0]
    x1 = x1r[0]
    x2 = x2r[0]
    y0 = y0r[0]
    y1 = y1r[0]
    y2 = y2r[0]

    sq1 = x0 * x0 + x1 * x1 + x2 * x2
    sq2 = y0 * y0 + y1 * y1 + y2 * y2

    s1h = _bf16_round(sq1)
    s1m = _bf16_round(sq1 - s1h)
    s1l = sq1 - s1h - s1m
    s2h = _bf16_round(sq2)
    s2m = _bf16_round(sq2 - s2h)
    s2l = sq2 - s2h - s2m
    onex = jnp.ones_like(x)
    x9 = jnp.concatenate([x * -2.0, onex, s1h, s1m, s1l],
                         axis=1).astype(jnp.bfloat16)
    y9 = jnp.concatenate(
        [y0, y1, y2, s2h, s2m, s2l,
         jnp.ones((3,) + y0.shape[1:], jnp.float32)],
        axis=0).astype(jnp.bfloat16)
    u0 = _bf16_round(x0) * -2.0
    u1 = _bf16_round(x1) * -2.0
    u2 = _bf16_round(x2) * -2.0
    v0 = _bf16_round(y0)
    v1 = _bf16_round(y1)
    v2 = _bf16_round(y2)

    d1col = None
    s2 = jnp.zeros((1, 1), jnp.float32)
    for c in range(m // mc):
        lo, hi = c * mc, (c + 1) * mc
        if c < n_mxu_chunks:
            d = jax.lax.dot_general(
                x9, y9[:, lo:hi], (((1,), (0,)), ((), ())),
                preferred_element_type=jnp.float32)
        else:
            d = (sq1 + sq2[:, lo:hi]) + u0 * v0[:, lo:hi]
            d = d + u1 * v1[:, lo:hi]
            d = d + u2 * v2[:, lo:hi]
        cmin = jnp.maximum(jnp.min(d, axis=0, keepdims=True), 0.0)
        d2r[0, :, lo:hi] = cmin
        s2 = s2 + jnp.sum(cmin, axis=1, keepdims=True)
        rmin = jnp.min(d, axis=1, keepdims=True)
        d1col = rmin if d1col is None else jnp.minimum(d1col, rmin)

    d1 = jnp.maximum(d1col, 0.0)
    d1r[0] = d1
    sr[0] = jnp.sum(d1, axis=0, keepdims=True) * inv1 + s2 * inv2


def kernel(pcs1, pcs2):
    B, N, D = pcs1.shape
    M = pcs2.shape[1]
    assert D == 3
    f32 = jnp.float32

    x0 = pcs1[:, :, 0:1]
    x1 = pcs1[:, :, 1:2]
    x2 = pcs1[:, :, 2:3]
    y0 = pcs2[:, :, 0][:, None, :]
    y1 = pcs2[:, :, 1][:, None, :]
    y2 = pcs2[:, :, 2][:, None, :]

    n_chunks = M // _MC
    n_mxu = int(round(_MXU_FRAC * n_chunks))
    body = functools.partial(_cd_kernel, m=M, mc=_MC, n_mxu_chunks=n_mxu,
                             inv1=1.0 / (B * N), inv2=1.0 / (B * M))

    d1, d2, psum = pl.pallas_call(
        body,
        grid=(B,),
        in_specs=[
            pl.BlockSpec((1, N, 3), lambda i: (i, 0, 0)),
            pl.BlockSpec((1, N, 1), lambda i: (i, 0, 0)),
            pl.BlockSpec((1, N, 1), lambda i: (i, 0, 0)),
            pl.BlockSpec((1, N, 1), lambda i: (i, 0, 0)),
            pl.BlockSpec((1, 1, M), lambda i: (i, 0, 0)),
            pl.BlockSpec((1, 1, M), lambda i: (i, 0, 0)),
            pl.BlockSpec((1, 1, M), lambda i: (i, 0, 0)),
        ],
        out_specs=[
            pl.BlockSpec((1, N, 1), lambda i: (i, 0, 0)),
            pl.BlockSpec((1, 1, M), lambda i: (i, 0, 0)),
            pl.BlockSpec((1, 1, 1), lambda i: (i, 0, 0)),
        ],
        out_shape=[
            jax.ShapeDtypeStruct((B, N, 1), f32),
            jax.ShapeDtypeStruct((B, 1, M), f32),
            jax.ShapeDtypeStruct((B, 1, 1), f32),
        ],
        compiler_params=pltpu.CompilerParams(
            dimension_semantics=("parallel",)),
    )(pcs1, x0, x1, x2, y0, y1, y2)

    mean = jnp.sum(psum)
    return mean, d1[:, :, 0], d2[:, 0, :]


# speedup vs baseline: 1.1008x; 1.0010x over previous
import functools

import jax
import jax.numpy as jnp
from jax.experimental import pallas as pl
from jax.experimental.pallas import tpu as pltpu

_MC = 1024
_MXU_FRAC = 1.0


def _bf16_round(v):
    return v.astype(jnp.bfloat16).astype(jnp.float32)


def _cd_kernel(xr, x0r, x1r, x2r, y0r, y1r, y2r, d1r, d2r, sr, *, m, mc,
               n_mxu_chunks, inv1, inv2):
    x = xr[0]
    x0 = x0r[0]
    x1 = x1r[0]
    x2 = x2r[0]
    y0 = y0r[0]
    y1 = y1r[0]
    y2 = y2r[0]

    sq1 = x0 * x0 + x1 * x1 + x2 * x2
    sq2 = y0 * y0 + y1 * y1 + y2 * y2

    s1h = _bf16_round(sq1)
    s1m = _bf16_round(sq1 - s1h)
    s1l = sq1 - s1h - s1m
    s2h = _bf16_round(sq2)
    s2m = _bf16_round(sq2 - s2h)
    s2l = sq2 - s2h - s2m
    onex = jnp.ones_like(x)
    x9 = jnp.concatenate([x * -2.0, onex, s1h, s1m, s1l],
                         axis=1).astype(jnp.bfloat16)
    y9 = jnp.concatenate(
        [y0, y1, y2, s2h, s2m, s2l,
         jnp.ones((3,) + y0.shape[1:], jnp.float32)],
        axis=0).astype(jnp.bfloat16)
    u0 = _bf16_round(x0) * -2.0
    u1 = _bf16_round(x1) * -2.0
    u2 = _bf16_round(x2) * -2.0
    v0 = _bf16_round(y0)
    v1 = _bf16_round(y1)
    v2 = _bf16_round(y2)

    d1col = None
    s2 = jnp.zeros((1, 1), jnp.float32)
    for c in range(m // mc):
        lo, hi = c * mc, (c + 1) * mc
        if c < n_mxu_chunks:
            d = jax.lax.dot_general(
                x9, y9[:, lo:hi], (((1,), (0,)), ((), ())),
                preferred_element_type=jnp.float32)
        else:
            d = (sq1 + sq2[:, lo:hi]) + u0 * v0[:, lo:hi]
            d = d + u1 * v1[:, lo:hi]
            d = d + u2 * v2[:, lo:hi]
        cmin = jnp.maximum(jnp.min(d, axis=0, keepdims=True), 0.0)
        d2r[0, :, lo:hi] = cmin
        s2 = s2 + jnp.sum(cmin, axis=1, keepdims=True)
        rmin = jnp.min(d, axis=1, keepdims=True)
        d1col = rmin if d1col is None else jnp.minimum(d1col, rmin)

    d1 = jnp.maximum(d1col, 0.0)
    d1r[0] = d1
    sr[0] = jnp.sum(d1, axis=0, keepdims=True) * inv1 + s2 * inv2


def kernel(pcs1, pcs2):
    B, N, D = pcs1.shape
    M = pcs2.shape[1]
    assert D == 3
    f32 = jnp.float32

    x0 = pcs1[:, :, 0:1]
    x1 = pcs1[:, :, 1:2]
    x2 = pcs1[:, :, 2:3]
    y0 = pcs2[:, :, 0][:, None, :]
    y1 = pcs2[:, :, 1][:, None, :]
    y2 = pcs2[:, :, 2][:, None, :]

    n_chunks = M // _MC
    n_mxu = int(round(_MXU_FRAC * n_chunks))
    body = functools.partial(_cd_kernel, m=M, mc=_MC, n_mxu_chunks=n_mxu,
                             inv1=1.0 / (B * N), inv2=1.0 / (B * M))

    d1, d2, psum = pl.pallas_call(
        body,
        grid=(B,),
        in_specs=[
            pl.BlockSpec((1, N, 3), lambda i: (i, 0, 0)),
            pl.BlockSpec((1, N, 1), lambda i: (i, 0, 0)),
            pl.BlockSpec((1, N, 1), lambda i: (i, 0, 0)),
            pl.BlockSpec((1, N, 1), lambda i: (i, 0, 0)),
            pl.BlockSpec((1, 1, M), lambda i: (i, 0, 0)),
            pl.BlockSpec((1, 1, M), lambda i: (i, 0, 0)),
            pl.BlockSpec((1, 1, M), lambda i: (i, 0, 0)),
        ],
        out_specs=[
            pl.BlockSpec((1, N, 1), lambda i: (i, 0, 0)),
            pl.BlockSpec((1, 1, M), lambda i: (i, 0, 0)),
            pl.BlockSpec((1, 1, 1), lambda i: (i, 0, 0)),
        ],
        out_shape=[
            jax.ShapeDtypeStruct((B, N, 1), f32),
            jax.ShapeDtypeStruct((B, 1, M), f32),
            jax.ShapeDtypeStruct((B, 1, 1), f32),
        ],
        compiler_params=pltpu.CompilerParams(
            dimension_semantics=("parallel",)),
    )(pcs1, x0, x1, x2, y0, y1, y2)

    mean = jnp.sum(psum)
    return mean, d1[:, :, 0], d2[:, 0, :]
